# Initial kernel scaffold; baseline (speedup 1.0000x reference)
import jax
import jax.numpy as jnp
from jax.experimental import pallas as pl


def kernel(x, Wr, Wblk):
    raise NotImplementedError("write your pallas kernel here")


# speedup vs baseline: 5.7675x; 5.7675x over previous
import functools

import jax
import jax.numpy as jnp
from jax.experimental import pallas as pl
from jax.experimental.pallas import tpu as pltpu


def _logits_kernel(x_ref, wr_ref, lg_ref):
    xb = x_ref[0]
    wr = wr_ref[0]
    lg_ref[0, 0, 0, :] = jnp.sum(xb * wr[None, :], axis=1)


def _route_kernel(topk, lg_ref, w_ref):
    b, s = lg_ref.shape
    lg = lg_ref[...]
    kbits = jax.lax.bitcast_convert_type(lg, jnp.int32)
    m = jnp.where(kbits >= 0, kbits, kbits ^ jnp.int32(0x7FFFFFFF))
    int_min = jnp.int32(-(2**31))

    def body(i, prefix):
        bit = jnp.int32(1) << (31 - i)
        cand = prefix | bit
        cand_s = cand ^ int_min
        cnt = jnp.sum((m >= cand_s).astype(jnp.int32), axis=1, keepdims=True)
        return jnp.where(cnt >= topk, cand, prefix)

    prefix = jax.lax.fori_loop(0, 32, body, jnp.zeros((b, 1), jnp.int32))
    thr = prefix ^ int_min

    gt = m > thr
    eq = m == thr
    cnt_gt = jnp.sum(gt.astype(jnp.int32), axis=1, keepdims=True)
    need = topk - cnt_gt

    tie = (s - 1) - jax.lax.broadcasted_iota(jnp.int32, (b, s), 1)

    def body2(i, p):
        bit = jnp.int32(1) << (12 - i)
        cand = p | bit
        cnt = jnp.sum((eq & (tie >= cand)).astype(jnp.int32), axis=1,
                      keepdims=True)
        return jnp.where(cnt >= need, cand, p)

    p2 = jax.lax.fori_loop(0, 13, body2, jnp.zeros((b, 1), jnp.int32))
    sel = gt | (eq & (tie >= p2))

    mx = jnp.max(lg, axis=1, keepdims=True)
    e = jnp.where(sel, jnp.exp(lg - mx), 0.0)
    w_ref[...] = e / jnp.sum(e, axis=1, keepdims=True)


def _block_kernel(x_ref, w_ref, wb_ref, o_ref):
    xb = x_ref[0]
    w = w_ref[0, 0, 0, :]
    z = (xb * w[:, None]).astype(jnp.bfloat16)
    y = jax.lax.dot_general(z, wb_ref[...], (((1,), (1,)), ((), ())),
                            preferred_element_type=jnp.float32)
    o_ref[0] = xb + y


def kernel(x, Wr, Wblk):
    b, s, d = x.shape
    topk = s // 2
    ts = min(1024, s)
    nj = s // ts

    lg4 = pl.pallas_call(
        _logits_kernel,
        grid=(b, nj),
        in_specs=[
            pl.BlockSpec((1, ts, d), lambda i, j: (i, j, 0)),
            pl.BlockSpec((1, d), lambda i, j: (0, 0)),
        ],
        out_specs=pl.BlockSpec((1, 1, 1, ts), lambda i, j: (i, j, 0, 0)),
        out_shape=jax.ShapeDtypeStruct((b, nj, 1, ts), jnp.float32),
        compiler_params=pltpu.CompilerParams(
            dimension_semantics=("parallel", "parallel")),
    )(x, Wr)

    lg = lg4.reshape(b, s)

    w = pl.pallas_call(
        functools.partial(_route_kernel, topk),
        out_shape=jax.ShapeDtypeStruct((b, s), jnp.float32),
    )(lg)

    w4 = w.reshape(b, nj, 1, ts)

    out = pl.pallas_call(
        _block_kernel,
        grid=(b, nj),
        in_specs=[
            pl.BlockSpec((1, ts, d), lambda i, j: (i, j, 0)),
            pl.BlockSpec((1, 1, 1, ts), lambda i, j: (i, j, 0, 0)),
            pl.BlockSpec((d, d), lambda i, j: (0, 0)),
        ],
        out_specs=pl.BlockSpec((1, ts, d), lambda i, j: (i, j, 0)),
        out_shape=jax.ShapeDtypeStruct((b, s, d), jnp.float32),
        compiler_params=pltpu.CompilerParams(
            dimension_semantics=("parallel", "parallel")),
    )(x, w4, Wblk.astype(jnp.bfloat16))

    return out
